# baseline (speedup 1.0000x reference)
import jax
import jax.numpy as jnp
from jax import lax
from jax.experimental import pallas as pl
from jax.experimental.pallas import tpu as pltpu
from jax.experimental.pallas import tpu_sc as plsc

NUM_LODS = 11
H = 1024
W = 1024
C = 9
B = 16384
CP = 16

NC, NS, L = 2, 16, 16
NW = NC * NS
B_PER_W = B // NW
NLINE = C * B_PER_W
CHUNK = 128
NCHUNK = NLINE // CHUNK
N_LINES_TOTAL = NUM_LODS * H * W * C // 16


def _sc_gather(table_hbm, bi_hbm, out_hbm, bi_v, lin_v, lane_v, lines_v, out_v, sem):
    wid = lax.axis_index("s") * NC + lax.axis_index("c")
    base = wid * B_PER_W

    pltpu.sync_copy(bi_hbm.at[pl.ds(base * 3, B_PER_W * 3)], bi_v)
    lane = lax.iota(jnp.int32, L)
    lane3 = lane * 3
    for i in range(B_PER_W // L):
        rows3 = lane3 + (i * L * 3)
        y = plsc.load_gather(bi_v, [rows3])
        x = plsc.load_gather(bi_v, [rows3 + 1])
        lod = plsc.load_gather(bi_v, [rows3 + 2])
        sy = lax.shift_right_logical(y, lod)
        sx = lax.shift_right_logical(x, lod)
        line0 = (
            lod * (C * 65536)
            + lax.shift_left(lax.shift_right_logical(sy, 3), 9)
            + lax.shift_left(lax.shift_right_logical(sx, 7), 6)
            + lax.shift_left(lax.bitwise_and(sy, 7), 3)
            + lax.bitwise_and(lax.shift_right_logical(sx, 4), 7)
        )
        lane_v[pl.ds(i * L, L)] = lax.bitwise_and(sx, 15)
        for c in range(C):
            lin_v[pl.ds(c * B_PER_W + i * L, L)] = line0 + (c << 16)

    copies = [
        pltpu.async_copy(
            table_hbm.at[lin_v.at[pl.ds(j * CHUNK, CHUNK)]],
            lines_v.at[pl.ds(j * CHUNK, CHUNK)],
            sem,
        )
        for j in range(NCHUNK)
    ]
    for cp in copies:
        cp.wait()

    for i in range(B_PER_W // L):
        lanes = lane_v[pl.ds(i * L, L)]
        rows = lane + (i * L)
        for c in range(C):
            v = plsc.load_gather(lines_v, [rows + c * B_PER_W, lanes])
            out_v[
                (c // 8) * 4 + (i // 8),
                pl.ds((c % 8) * 128 + (i % 8) * L, L),
            ] = v

    for cg in range(2):
        for bg in range(4):
            pltpu.sync_copy(
                out_v.at[cg * 4 + bg],
                out_hbm.at[pl.ds((cg * 128 + wid * 4 + bg) * 1024, 1024)],
            )


def kernel(batch_index, lod_cache):
    t = lod_cache.transpose(0, 3, 1, 2)
    t = t.reshape(NUM_LODS, C, H // 8, 8, W // 128, 128)
    t = t.transpose(0, 1, 2, 4, 3, 5)
    table = t.reshape(N_LINES_TOTAL, 16)
    mesh = plsc.VectorSubcoreMesh(
        core_axis_name="c", subcore_axis_name="s", num_cores=NC, num_subcores=NS
    )
    run = pl.kernel(
        _sc_gather,
        out_type=jax.ShapeDtypeStruct((B * CP,), jnp.float32),
        mesh=mesh,
        compiler_params=pltpu.CompilerParams(
            needs_layout_passes=False, use_tc_tiling_on_sc=False
        ),
        scratch_types=[
            pltpu.VMEM((B_PER_W * 3,), jnp.int32),
            pltpu.VMEM((NLINE,), jnp.int32),
            pltpu.VMEM((B_PER_W,), jnp.int32),
            pltpu.VMEM((NLINE, 16), jnp.float32),
            pltpu.VMEM((8, 1024), jnp.float32),
            pltpu.SemaphoreType.DMA,
        ],
    )
    o = run(table, batch_index.reshape(B * 3))
    o = o.reshape(CP // 8, B // 128, 8, 128).transpose(1, 3, 0, 2).reshape(B, CP)
    return o[:, :C]


# speedup vs baseline: 1.1038x; 1.1038x over previous
import jax
import jax.numpy as jnp
from jax import lax
from jax.experimental import pallas as pl
from jax.experimental.pallas import tpu as pltpu
from jax.experimental.pallas import tpu_sc as plsc

NUM_LODS = 11
H = 1024
W = 1024
C = 9
B = 16384
CP = 16

NC, NS, L = 2, 16, 16
NW = NC * NS
B_PER_W = B // NW
NELEM = C * B_PER_W
CHUNK = 128
NCHUNK = NELEM // CHUNK
N_WORDS_TOTAL = NUM_LODS * H * W * C


def _sc_gather(table_hbm, bi_hbm, out_hbm, bi_v, idx_v, tmp_v, out_v, sem):
    wid = lax.axis_index("s") * NC + lax.axis_index("c")
    base = wid * B_PER_W

    pltpu.sync_copy(bi_hbm.at[pl.ds(base * 3, B_PER_W * 3)], bi_v)
    lane = lax.iota(jnp.int32, L)
    lane3 = lane * 3
    for i in range(B_PER_W // L):
        rows3 = lane3 + (i * L * 3)
        y = plsc.load_gather(bi_v, [rows3])
        x = plsc.load_gather(bi_v, [rows3 + 1])
        lod = plsc.load_gather(bi_v, [rows3 + 2])
        sy = lax.shift_right_logical(y, lod)
        sx = lax.shift_right_logical(x, lod)
        w0 = (
            lod * (C * 1048576)
            + lax.shift_left(lax.shift_right_logical(sy, 3), 13)
            + lax.shift_left(lax.shift_right_logical(sx, 7), 10)
            + lax.shift_left(lax.bitwise_and(sy, 7), 7)
            + lax.bitwise_and(sx, 127)
        )
        for c in range(C):
            idx_v[pl.ds(c * B_PER_W + i * L, L)] = w0 + (c << 20)

    copies = [
        pltpu.async_copy(
            table_hbm.at[idx_v.at[pl.ds(j * CHUNK, CHUNK)]],
            tmp_v.at[pl.ds(j * CHUNK, CHUNK)],
            sem,
        )
        for j in range(NCHUNK)
    ]
    for cp in copies:
        cp.wait()

    for i in range(B_PER_W // L):
        for c in range(C):
            v = tmp_v[pl.ds(c * B_PER_W + i * L, L)]
            out_v[
                c // 8,
                pl.ds((i // 8) * 1024 + (c % 8) * 128 + (i % 8) * L, L),
            ] = v

    ocopies = [
        pltpu.async_copy(
            out_v.at[cg],
            out_hbm.at[pl.ds((cg * 128 + wid * 4) * 1024, 4096)],
            sem,
        )
        for cg in range(2)
    ]
    for cp in ocopies:
        cp.wait()


def kernel(batch_index, lod_cache):
    t = lod_cache.transpose(0, 3, 1, 2)
    t = t.reshape(NUM_LODS, C, H // 8, 8, W // 128, 128)
    t = t.transpose(0, 1, 2, 4, 3, 5)
    table = t.reshape(N_WORDS_TOTAL)
    mesh = plsc.VectorSubcoreMesh(
        core_axis_name="c", subcore_axis_name="s", num_cores=NC, num_subcores=NS
    )
    run = pl.kernel(
        _sc_gather,
        out_type=jax.ShapeDtypeStruct((B * CP,), jnp.float32),
        mesh=mesh,
        compiler_params=pltpu.CompilerParams(
            needs_layout_passes=False, use_tc_tiling_on_sc=False
        ),
        scratch_types=[
            pltpu.VMEM((B_PER_W * 3,), jnp.int32),
            pltpu.VMEM((NELEM,), jnp.int32),
            pltpu.VMEM((NELEM,), jnp.float32),
            pltpu.VMEM((2, 4096), jnp.float32),
            pltpu.SemaphoreType.DMA,
        ],
    )
    o = run(table, batch_index.reshape(B * 3))
    o = o.reshape(CP // 8, B // 128, 8, 128).transpose(1, 3, 0, 2).reshape(B, CP)
    return o[:, :C]
